# baseline (speedup 1.0000x reference)
import jax
import jax.numpy as jnp
from jax import lax
from jax.experimental import pallas as pl
from jax.experimental.pallas import tpu as pltpu

_BLK = 256


def _model_body(adj_ref, scal_ref, arr_ref, dep_ref, ihtm_ref, vcol_ref,
                vrow_ref, we1_ref, be1_ref, we2_ref, be2_ref, gw_ref,
                gas_ref, gad_ref, gb_ref, wsel_ref, out_ref, buf_ref,
                sem_ref):
    f32 = jnp.float32
    bf16 = jnp.bfloat16
    log2e = 1.4426950408889634

    n = arr_ref.shape[0]
    nb = n // _BLK

    def start_copy(b):
        pltpu.make_async_copy(
            adj_ref.at[:, pl.ds(b * _BLK, _BLK)],
            buf_ref.at[b % 2], sem_ref.at[b % 2]).start()

    def wait_copy(b):
        pltpu.make_async_copy(
            adj_ref.at[:, pl.ds(b * _BLK, _BLK)],
            buf_ref.at[b % 2], sem_ref.at[b % 2]).wait()

    start_copy(0)
    start_copy(1)

    tsf = scal_ref[0, 0]
    cc = scal_ref[0, 1]
    arr = arr_ref[...]
    dep = dep_ref[...]
    ihtm = ihtm_ref[...]
    vcol = vcol_ref[...]
    vrow_big = vrow_ref[...] * 1e30

    prog = (tsf - arr) / (dep - arr)
    x = prog * we1_ref[0:1, :] + ihtm * we1_ref[1:2, :] + be1_ref[...]
    x = jnp.dot(x, we2_ref[...], preferred_element_type=f32) + be2_ref[...]

    ones_b = jnp.ones((n, 1), bf16)
    nlayers = gw_ref.shape[0]

    def attn_coeffs(hp, l):
        gas_l = gas_ref[l:l + 1, :] * log2e
        gad_l = gad_ref[l:l + 1, :] * log2e
        a_s_row = lax.dot_general(gas_l, hp, (((1,), (1,)), ((), ())),
                                  preferred_element_type=f32)
        a_s_col = lax.dot_general(hp, gas_l, (((1,), (1,)), ((), ())),
                                  preferred_element_type=f32)
        a_d_col = lax.dot_general(hp, gad_l, (((1,), (1,)), ((), ())),
                                  preferred_element_type=f32)
        es = a_s_col + a_d_col
        es = jnp.maximum(es, 0.2 * es)
        return a_s_row, a_s_row.astype(bf16), 0.2 * a_s_row, a_d_col, es

    def attn_block(plog, plog_b, hp, hpo_b, a_s_row, a_s_row_b, asr2,
                   a_d_col, es, gb_l, lo, w):
        adc = a_d_col[lo:lo + w, :]
        esb = es[lo:lo + w, :]
        ms = jnp.max(a_s_row_b + plog_b, axis=1,
                     keepdims=True).astype(f32)
        mr = adc + ms
        m = jnp.maximum(jnp.maximum(mr, 0.2 * mr), esb)
        u = (adc - m) + a_s_row
        v = (0.2 * adc - m) + asr2
        ex = jnp.exp2(jnp.maximum(u, v) + plog).astype(bf16)
        exs = jnp.exp2(esb - m)
        h = hpo_b.shape[1] - 1
        nd = jnp.dot(ex, hpo_b, preferred_element_type=f32)
        num = nd[:, :h]
        den = nd[:, h:]
        return (num + exs * hp[lo:lo + w, :h]) \
            * (1.0 / (den + exs + 1e-16)) + gb_l

    hp = jnp.dot(x, gw_ref[0], preferred_element_type=f32)
    hpo_b = jnp.concatenate([hp.astype(bf16), ones_b], axis=1)
    a_s_row, a_s_row_b, asr2, a_d_col, es = attn_coeffs(hp, 0)
    gb_l = gb_ref[0:1, :]

    plog_blocks = []
    plogb_blocks = []
    h_blocks = []
    for b in range(nb):
        wait_copy(b)
        at_blk = jnp.transpose(buf_ref[b % 2])
        if b + 2 < nb:
            start_copy(b + 2)
        plog = at_blk * vrow_big - 1e30
        plog_b = plog.astype(bf16)
        plog_blocks.append(plog)
        plogb_blocks.append(plog_b)
        o = attn_block(plog, plog_b, hp, hpo_b, a_s_row, a_s_row_b,
                       asr2, a_d_col, es, gb_l, b * _BLK, _BLK)
        h_blocks.append(jnp.maximum(o, 0.0))
    h = jnp.concatenate(h_blocks, axis=0)

    for l in range(1, nlayers):
        hp = jnp.dot(h, gw_ref[l], preferred_element_type=f32)
        hpo_b = jnp.concatenate([hp.astype(bf16), ones_b], axis=1)
        a_s_row, a_s_row_b, asr2, a_d_col, es = attn_coeffs(hp, l)
        gb_l = gb_ref[l:l + 1, :]
        h_blocks = []
        for b in range(nb):
            o = attn_block(plog_blocks[b], plogb_blocks[b], hp, hpo_b,
                           a_s_row, a_s_row_b, asr2, a_d_col, es, gb_l,
                           b * _BLK, _BLK)
            h_blocks.append(jnp.maximum(o, 0.0) if l < nlayers - 1 else o)
        h = jnp.concatenate(h_blocks, axis=0)

    x = x + h
    mu = jnp.mean(x, axis=1, keepdims=True)
    xc = x - mu
    var = jnp.mean(xc * xc, axis=1, keepdims=True)
    xn = xc * lax.rsqrt(var + 1e-5)
    logit = jnp.dot(xn, wsel_ref[...], preferred_element_type=f32) + cc
    out_ref[...] = vcol / (1.0 + jnp.exp(-logit))


def kernel(adj_matrix, timestep, arrival, departure, is_hard_to_match,
           total_timesteps, mask, W_emb1, b_emb1, W_emb2, b_emb2, gat_W,
           gat_att_src, gat_att_dst, gat_bias, W_sel, b_sel):
    n = adj_matrix.shape[0]
    hdim = W_emb2.shape[0]
    f32 = jnp.float32

    tsf = jnp.asarray(timestep, f32)
    ttf = jnp.asarray(total_timesteps, f32)
    cc = (tsf / ttf) * W_sel[hdim, 0] + b_sel[0]
    scal = jnp.stack([tsf, cc]).reshape(1, 2)

    vcol = (mask > 0).astype(f32).reshape(n, 1)
    vrow = vcol.reshape(1, n)

    vmem = pl.BlockSpec(memory_space=pltpu.MemorySpace.VMEM)
    out = pl.pallas_call(
        _model_body,
        out_shape=jax.ShapeDtypeStruct((n, 1), f32),
        in_specs=[pl.BlockSpec(memory_space=pl.ANY)] + [vmem] * 15,
        out_specs=vmem,
        scratch_shapes=[
            pltpu.VMEM((2, n, _BLK), f32),
            pltpu.SemaphoreType.DMA((2,)),
        ],
        compiler_params=pltpu.CompilerParams(
            vmem_limit_bytes=128 * 1024 * 1024),
    )(adj_matrix, scal, arrival.reshape(n, 1), departure.reshape(n, 1),
      is_hard_to_match.reshape(n, 1), vcol, vrow, W_emb1,
      b_emb1.reshape(1, hdim), W_emb2, b_emb2.reshape(1, hdim), gat_W,
      gat_att_src, gat_att_dst, gat_bias, W_sel[:hdim, :])
    return out


# speedup vs baseline: 1.0122x; 1.0122x over previous
import jax
import jax.numpy as jnp
from jax import lax
from jax.experimental import pallas as pl
from jax.experimental.pallas import tpu as pltpu

_BLK = 512


def _model_body(adj_ref, scal_ref, arr_ref, dep_ref, ihtm_ref, vcol_ref,
                vrow_ref, we1_ref, be1_ref, we2_ref, be2_ref, gw_ref,
                gas_ref, gad_ref, gb_ref, wsel_ref, out_ref, buf_ref,
                sem_ref):
    f32 = jnp.float32
    bf16 = jnp.bfloat16
    log2e = 1.4426950408889634

    n = arr_ref.shape[0]
    nb = n // _BLK

    def start_copy(b):
        pltpu.make_async_copy(
            adj_ref.at[:, pl.ds(b * _BLK, _BLK)],
            buf_ref.at[b % 2], sem_ref.at[b % 2]).start()

    def wait_copy(b):
        pltpu.make_async_copy(
            adj_ref.at[:, pl.ds(b * _BLK, _BLK)],
            buf_ref.at[b % 2], sem_ref.at[b % 2]).wait()

    start_copy(0)
    start_copy(1)

    tsf = scal_ref[0, 0]
    cc = scal_ref[0, 1]
    arr = arr_ref[...]
    dep = dep_ref[...]
    ihtm = ihtm_ref[...]
    vcol = vcol_ref[...]
    vrow_big = vrow_ref[...] * 1e30

    prog = (tsf - arr) / (dep - arr)
    x = prog * we1_ref[0:1, :] + ihtm * we1_ref[1:2, :] + be1_ref[...]
    x = jnp.dot(x, we2_ref[...], preferred_element_type=f32) + be2_ref[...]

    ones_b = jnp.ones((n, 1), bf16)
    nlayers = gw_ref.shape[0]

    def attn_coeffs(hp, l):
        gas_l = gas_ref[l:l + 1, :] * log2e
        gad_l = gad_ref[l:l + 1, :] * log2e
        a_s_row = lax.dot_general(gas_l, hp, (((1,), (1,)), ((), ())),
                                  preferred_element_type=f32)
        a_s_col = lax.dot_general(hp, gas_l, (((1,), (1,)), ((), ())),
                                  preferred_element_type=f32)
        a_d_col = lax.dot_general(hp, gad_l, (((1,), (1,)), ((), ())),
                                  preferred_element_type=f32)
        es = a_s_col + a_d_col
        es = jnp.maximum(es, 0.2 * es)
        return a_s_row, a_s_row.astype(bf16), 0.2 * a_s_row, a_d_col, es

    def attn_block(plog, plog_b, hp, hpo_b, a_s_row, a_s_row_b, asr2,
                   a_d_col, es, gb_l, lo, w):
        adc = a_d_col[lo:lo + w, :]
        esb = es[lo:lo + w, :]
        ms = jnp.max(a_s_row_b + plog_b, axis=1,
                     keepdims=True).astype(f32)
        mr = adc + ms
        m = jnp.maximum(jnp.maximum(mr, 0.2 * mr), esb)
        u = (adc - m) + a_s_row
        v = (0.2 * adc - m) + asr2
        ex = jnp.exp2(jnp.maximum(u, v) + plog).astype(bf16)
        exs = jnp.exp2(esb - m)
        h = hpo_b.shape[1] - 1
        nd = jnp.dot(ex, hpo_b, preferred_element_type=f32)
        num = nd[:, :h]
        den = nd[:, h:]
        return (num + exs * hp[lo:lo + w, :h]) \
            * (1.0 / (den + exs + 1e-16)) + gb_l

    hp = jnp.dot(x, gw_ref[0], preferred_element_type=f32)
    hpo_b = jnp.concatenate([hp.astype(bf16), ones_b], axis=1)
    a_s_row, a_s_row_b, asr2, a_d_col, es = attn_coeffs(hp, 0)
    gb_l = gb_ref[0:1, :]

    plog_blocks = []
    plogb_blocks = []
    h_blocks = []
    for b in range(nb):
        wait_copy(b)
        at_blk = jnp.transpose(buf_ref[b % 2])
        if b + 2 < nb:
            start_copy(b + 2)
        plog = at_blk * vrow_big - 1e30
        plog_b = plog.astype(bf16)
        plog_blocks.append(plog)
        plogb_blocks.append(plog_b)
        o = attn_block(plog, plog_b, hp, hpo_b, a_s_row, a_s_row_b,
                       asr2, a_d_col, es, gb_l, b * _BLK, _BLK)
        h_blocks.append(jnp.maximum(o, 0.0))
    h = jnp.concatenate(h_blocks, axis=0)

    for l in range(1, nlayers):
        hp = jnp.dot(h, gw_ref[l], preferred_element_type=f32)
        hpo_b = jnp.concatenate([hp.astype(bf16), ones_b], axis=1)
        a_s_row, a_s_row_b, asr2, a_d_col, es = attn_coeffs(hp, l)
        gb_l = gb_ref[l:l + 1, :]
        h_blocks = []
        for b in range(nb):
            o = attn_block(plog_blocks[b], plogb_blocks[b], hp, hpo_b,
                           a_s_row, a_s_row_b, asr2, a_d_col, es, gb_l,
                           b * _BLK, _BLK)
            h_blocks.append(jnp.maximum(o, 0.0) if l < nlayers - 1 else o)
        h = jnp.concatenate(h_blocks, axis=0)

    x = x + h
    mu = jnp.mean(x, axis=1, keepdims=True)
    xc = x - mu
    var = jnp.mean(xc * xc, axis=1, keepdims=True)
    xn = xc * lax.rsqrt(var + 1e-5)
    logit = jnp.dot(xn, wsel_ref[...], preferred_element_type=f32) + cc
    out_ref[...] = vcol / (1.0 + jnp.exp(-logit))


def kernel(adj_matrix, timestep, arrival, departure, is_hard_to_match,
           total_timesteps, mask, W_emb1, b_emb1, W_emb2, b_emb2, gat_W,
           gat_att_src, gat_att_dst, gat_bias, W_sel, b_sel):
    n = adj_matrix.shape[0]
    hdim = W_emb2.shape[0]
    f32 = jnp.float32

    tsf = jnp.asarray(timestep, f32)
    ttf = jnp.asarray(total_timesteps, f32)
    cc = (tsf / ttf) * W_sel[hdim, 0] + b_sel[0]
    scal = jnp.stack([tsf, cc]).reshape(1, 2)

    vcol = (mask > 0).astype(f32).reshape(n, 1)
    vrow = vcol.reshape(1, n)

    vmem = pl.BlockSpec(memory_space=pltpu.MemorySpace.VMEM)
    out = pl.pallas_call(
        _model_body,
        out_shape=jax.ShapeDtypeStruct((n, 1), f32),
        in_specs=[pl.BlockSpec(memory_space=pl.ANY)] + [vmem] * 15,
        out_specs=vmem,
        scratch_shapes=[
            pltpu.VMEM((2, n, _BLK), f32),
            pltpu.SemaphoreType.DMA((2,)),
        ],
        compiler_params=pltpu.CompilerParams(
            vmem_limit_bytes=128 * 1024 * 1024),
    )(adj_matrix, scal, arrival.reshape(n, 1), departure.reshape(n, 1),
      is_hard_to_match.reshape(n, 1), vcol, vrow, W_emb1,
      b_emb1.reshape(1, hdim), W_emb2, b_emb2.reshape(1, hdim), gat_W,
      gat_att_src, gat_att_dst, gat_bias, W_sel[:hdim, :])
    return out
